# Initial kernel scaffold; baseline (speedup 1.0000x reference)
import jax
import jax.numpy as jnp
from jax.experimental import pallas as pl


def kernel(user_code, item_code, user_occupation, item_timestamp_rank, user_emb, item_emb, occ_emb, user_temp_emb, temp_emb, user_bias, item_bias, bias):
    raise NotImplementedError("write your pallas kernel here")


# speedup vs baseline: 21.9002x; 21.9002x over previous
import functools

import jax
import jax.numpy as jnp
from jax import lax
from jax.experimental import pallas as pl
from jax.experimental.pallas import tpu as pltpu
from jax.experimental.pallas import tpu_sc as plsc

N_USERS = 100000
N_ITEMS = 100000
N_OCC = 64
MAX_TS = 1024
D = 32
B = 16384
L = 50

NC = 2
NS = 16
NW = NC * NS
LANES = 16

ROWS_W = B // NW
CHUNK = 32
N_CHUNKS = ROWS_W // CHUNK
ITEMS_CHUNK = CHUNK * L
GW = 40
NG = ITEMS_CHUNK // GW
IDX_ROWS = B * L // GW


def _body(user_code, item_code_r, user_occ, ts_rank,
          user_emb, item_emb, occ_emb, user_temp_emb, temp_emb,
          user_bias, item_bias, bias,
          out_hbm,
          idx_c, uc_all, oc_all, ts_all,
          rows_2d, ib_v, ue_2d, oe_2d, ute_2d, te_2d, ub_v, bias_sv, rc_row,
          out_v, sem):
  wid = lax.axis_index("s") * NC + lax.axis_index("c")
  base_w = wid * ROWS_W

  pltpu.sync_copy(user_code.at[pl.ds(base_w, ROWS_W)], uc_all)
  pltpu.sync_copy(user_occ.at[pl.ds(base_w, ROWS_W)], oc_all)
  pltpu.sync_copy(ts_rank.at[pl.ds(base_w, ROWS_W)], ts_all)
  pltpu.sync_copy(bias, bias_sv.at[pl.ds(0, 1)])

  iota = lax.broadcasted_iota(jnp.int32, (LANES,), 0)
  lane15 = iota == (LANES - 1)
  zeros16 = jnp.zeros((LANES,), jnp.int32)
  bias_vec = plsc.load_gather(bias_sv, [zeros16])

  def chunk_body(c, carry):
    base = base_w + c * CHUNK

    pltpu.sync_copy(
        item_code_r.at[pl.ds(wid * (IDX_ROWS // NW) + c * NG, NG)], idx_c)
    copies = []
    for j in range(NG):
      copies.append(pltpu.async_copy(
          item_emb.at[idx_c.at[j]], rows_2d.at[pl.ds(j * GW, GW)], sem))
    for j in range(NG):
      copies.append(pltpu.async_copy(
          item_bias.at[idx_c.at[j]], ib_v.at[pl.ds(j * GW, GW)], sem))
    uc = uc_all.at[pl.ds(c * CHUNK, CHUNK)]
    copies.append(pltpu.async_copy(user_emb.at[uc], ue_2d, sem))
    copies.append(pltpu.async_copy(user_temp_emb.at[uc], ute_2d, sem))
    copies.append(pltpu.async_copy(user_bias.at[uc], ub_v, sem))
    copies.append(pltpu.async_copy(
        occ_emb.at[oc_all.at[pl.ds(c * CHUNK, CHUNK)]], oe_2d, sem))
    copies.append(pltpu.async_copy(
        temp_emb.at[ts_all.at[pl.ds(c * CHUNK, CHUNK)]], te_2d, sem))
    for cp in copies:
      cp.wait()

    def row_body(r, _):
      q0 = ue_2d[r, pl.ds(0, LANES)] + oe_2d[r, pl.ds(0, LANES)]
      q1 = ue_2d[r, pl.ds(LANES, LANES)] + oe_2d[r, pl.ds(LANES, LANES)]
      tp = (ute_2d[r, pl.ds(0, LANES)] * te_2d[r, pl.ds(0, LANES)] +
            ute_2d[r, pl.ds(LANES, LANES)] * te_2d[r, pl.ds(LANES, LANES)])
      plsc.store_scatter(rc_row, [jnp.full((LANES,), 0, jnp.int32) + r],
                         plsc.cumsum(tp), mask=lane15)

      def item_body(l, _):
        i = r * L + l
        p = (rows_2d[i, pl.ds(0, LANES)] * q0 +
             rows_2d[i, pl.ds(LANES, LANES)] * q1)
        plsc.store_scatter(out_v, [jnp.full((LANES,), 0, jnp.int32) + i],
                           plsc.cumsum(p), mask=lane15)
        return 0

      lax.fori_loop(0, L, item_body, 0, unroll=8)
      return 0

    lax.fori_loop(0, CHUNK, row_body, 0, unroll=False)

    def fin_body(k, _):
      offs = k * LANES
      rows = (offs + iota) // L
      vals = (out_v[pl.ds(offs, LANES)] + ib_v[pl.ds(offs, LANES)] +
              plsc.load_gather(rc_row, [rows]) +
              plsc.load_gather(ub_v, [rows]) + bias_vec)
      out_v[pl.ds(offs, LANES)] = vals
      return 0

    lax.fori_loop(0, ITEMS_CHUNK // LANES, fin_body, 0, unroll=4)

    pltpu.sync_copy(out_v, out_hbm.at[pl.ds(base * L, CHUNK * L)])
    return carry

  lax.fori_loop(0, N_CHUNKS, chunk_body, 0, unroll=False)


@jax.jit
def _run(user_code, item_code_r, user_occ, ts_rank,
         user_emb, item_emb, occ_emb, user_temp_emb, temp_emb,
         user_bias, item_bias, bias):
  mesh = plsc.VectorSubcoreMesh(core_axis_name="c", subcore_axis_name="s",
                                num_cores=NC, num_subcores=NS)
  f = functools.partial(
      pl.kernel,
      out_type=jax.ShapeDtypeStruct((B * L,), jnp.float32),
      mesh=mesh,
      compiler_params=pltpu.CompilerParams(needs_layout_passes=False,
                                           use_tc_tiling_on_sc=False),
      scratch_types=[
          pltpu.VMEM((NG, GW), jnp.int32),
          pltpu.VMEM((ROWS_W,), jnp.int32),
          pltpu.VMEM((ROWS_W,), jnp.int32),
          pltpu.VMEM((ROWS_W,), jnp.int32),
          pltpu.VMEM((ITEMS_CHUNK, D), jnp.float32),
          pltpu.VMEM((ITEMS_CHUNK,), jnp.float32),
          pltpu.VMEM((CHUNK, D), jnp.float32),
          pltpu.VMEM((CHUNK, D), jnp.float32),
          pltpu.VMEM((CHUNK, D), jnp.float32),
          pltpu.VMEM((CHUNK, D), jnp.float32),
          pltpu.VMEM((CHUNK,), jnp.float32),
          pltpu.VMEM((LANES,), jnp.float32),
          pltpu.VMEM((CHUNK,), jnp.float32),
          pltpu.VMEM((ITEMS_CHUNK,), jnp.float32),
          pltpu.SemaphoreType.DMA,
      ],
  )(_body)
  return f(user_code, item_code_r, user_occ, ts_rank,
           user_emb, item_emb, occ_emb, user_temp_emb, temp_emb,
           user_bias, item_bias, bias)


def kernel(user_code, item_code, user_occupation, item_timestamp_rank,
           user_emb, item_emb, occ_emb, user_temp_emb, temp_emb,
           user_bias, item_bias, bias):
  item_code_r = item_code.reshape(IDX_ROWS, GW)
  out = _run(user_code, item_code_r, user_occupation, item_timestamp_rank,
             user_emb, item_emb, occ_emb, user_temp_emb, temp_emb,
             user_bias, item_bias, bias)
  return out.reshape(B, L)
